# Initial kernel scaffold; baseline (speedup 1.0000x reference)
import jax
import jax.numpy as jnp
from jax.experimental import pallas as pl


def kernel(x, edge_index, edge_attr, batch, atom_emb, bond_emb, vn0, gin_W1, gin_b1, gin_bn1_g, gin_bn1_b, gin_W2, gin_b2, gin_eps, bn_g, bn_b, vn_W1, vn_b1, vn_bn1_g, vn_bn1_b, vn_W2, vn_b2, vn_bn2_g, vn_bn2_b, sag_W, sag_b):
    raise NotImplementedError("write your pallas kernel here")


# speedup vs baseline: 1.0001x; 1.0001x over previous
import math

import jax
import jax.numpy as jnp
from jax.experimental import pallas as pl

EMB = 128
N0 = 10000
E = 320000
B = 8
NPER0 = N0 // B
BN_EPS = 1e-5
NUM_LAYERS = 3


def _bn(x, g, b):
    return g * x / jnp.sqrt(1.0 + BN_EPS) + b


def kernel(x, edge_index, edge_attr, batch, atom_emb, bond_emb, vn0,
           gin_W1, gin_b1, gin_bn1_g, gin_bn1_b, gin_W2, gin_b2, gin_eps,
           bn_g, bn_b, vn_W1, vn_b1, vn_bn1_g, vn_bn1_b, vn_W2, vn_b2,
           vn_bn2_g, vn_bn2_b, sag_W, sag_b):
    eattr = bond_emb[0][edge_attr[:, 0]] + bond_emb[1][edge_attr[:, 1]] + bond_emb[2][edge_attr[:, 2]]
    h0 = jnp.zeros((N0, EMB), jnp.float32)
    for f in range(9):
        h0 = h0 + atom_emb[f][x[:, f]]
    vn = vn0[jnp.zeros((B,), dtype=jnp.int32)]
    src = edge_index[0]
    dst = edge_index[1]
    valid = jnp.ones((E,), jnp.float32)
    N = N0
    n_per = NPER0
    bcur = batch
    h_list = [h0]
    b_list = [batch]
    for layer in range(NUM_LAYERS):
        h_in = h_list[layer] + vn[bcur]
        h_list[layer] = h_in
        msg = jax.nn.relu(h_in[src] + eattr) * valid[:, None]
        agg = jax.ops.segment_sum(msg, dst, num_segments=N)
        z = (1.0 + gin_eps[layer]) * h_in + agg
        z = _bn(z @ gin_W1[layer] + gin_b1[layer], gin_bn1_g[layer], gin_bn1_b[layer])
        z = jax.nn.relu(z)
        h = z @ gin_W2[layer] + gin_b2[layer]
        h = _bn(h, bn_g[layer], bn_b[layer])
        if layer < NUM_LAYERS - 1:
            h = jax.nn.relu(h)
            vt = jax.ops.segment_sum(h_in, bcur, num_segments=B) + vn
            t = _bn(vt @ vn_W1[layer] + vn_b1[layer], vn_bn1_g[layer], vn_bn1_b[layer])
            t = jax.nn.relu(t)
            t = _bn(t @ vn_W2[layer] + vn_b2[layer], vn_bn2_g[layer], vn_bn2_b[layer])
            vn = jax.nn.relu(t)
            xw = h @ sag_W
            deg = jax.ops.segment_sum(valid, dst, num_segments=N) + 1.0
            dinv = deg ** -0.5
            norm = dinv[src] * dinv[dst] * valid
            sc = jax.ops.segment_sum(norm[:, None] * xw[src], dst, num_segments=N)
            sc = sc + (dinv ** 2)[:, None] * xw + sag_b
            score = jnp.tanh(sc.reshape(-1))
            k = int(math.ceil(0.5 * n_per))
            _, top_i = jax.lax.top_k(score.reshape(B, n_per), k)
            perm = (top_i + (jnp.arange(B) * n_per)[:, None]).reshape(-1)
            h = h[perm] * score[perm][:, None]
            Nnew = B * k
            inv = jnp.full((N,), -1, dtype=jnp.int32).at[perm].set(jnp.arange(Nnew, dtype=jnp.int32))
            vr = inv[src]
            vc = inv[dst]
            valid = valid * (vr >= 0).astype(jnp.float32) * (vc >= 0).astype(jnp.float32)
            src = jnp.where(vr >= 0, vr, 0)
            dst = jnp.where(vc >= 0, vc, 0)
            bcur = jnp.repeat(jnp.arange(B), k)
            N = Nnew
            n_per = k
        h_list.append(h)
        b_list.append(bcur)
    return tuple(h_list) + tuple(b_list)


# speedup vs baseline: 1.4559x; 1.4558x over previous
import functools
import math

import jax
import jax.numpy as jnp
from jax.experimental import pallas as pl

EMB = 128
N0 = 10000
E = 320000
B = 8
NPER0 = N0 // B
BN_EPS = 1e-5
NUM_LAYERS = 3

LAYER_N = [10000, 5000, 2504]
LAYER_NPER = [1250, 625, 313]
LAYER_K = [625, 313]


def _npad(n):
    return 1280 * ((n + 1 + 1279) // 1280)


NP_L = [_npad(n) for n in LAYER_N]
NEG_BIG = -3.0e38


def _bn(x, g, b):
    return g * x / jnp.sqrt(1.0 + BN_EPS) + b


def _split3(t):
    def trunc(v):
        return jax.lax.bitcast_convert_type(
            jax.lax.bitcast_convert_type(v, jnp.uint32) & jnp.uint32(0xFFFF0000),
            jnp.float32)
    p1 = trunc(t)
    r = t - p1
    p2 = trunc(r)
    p3 = r - p2
    return jnp.stack([p1, p2, p3])


def _exact_sel(oh, t1, t2, t3):
    d1 = jnp.dot(oh, t1, preferred_element_type=jnp.float32)
    d2 = jnp.dot(oh, t2, preferred_element_type=jnp.float32)
    d3 = jnp.dot(oh, t3, preferred_element_type=jnp.float32)
    return (d1 + d2) + d3


def _atom_body(x_ref, emb_ref, o_ref):
    o_ref[...] = jnp.zeros((512, EMB), jnp.float32)
    for f in range(9):
        xc = x_ref[:, f:f + 1]
        oh = (jax.lax.broadcasted_iota(jnp.int32, (512, 16), 1) == xc
              ).astype(jnp.float32)
        g = _exact_sel(oh, emb_ref[0, f], emb_ref[1, f], emb_ref[2, f])
        o_ref[...] = o_ref[...] + g


def _atom_encode(xp, atom_emb3):
    npr = xp.shape[0]
    return pl.pallas_call(
        _atom_body,
        grid=(npr // 512,),
        in_specs=[pl.BlockSpec((512, 16), lambda i: (i, 0)),
                  pl.BlockSpec((3, 9, 16, EMB), lambda i: (0, 0, 0, 0))],
        out_specs=pl.BlockSpec((512, EMB), lambda i: (i, 0)),
        out_shape=jax.ShapeDtypeStruct((npr, EMB), jnp.float32),
    )(xp, atom_emb3)


def _t3_body(be_ref, o_ref):
    r = jax.lax.broadcasted_iota(jnp.int32, (512, 1), 0)
    a0 = r >> 6
    a1 = (r >> 3) & 7
    a2 = r & 7
    lane = jax.lax.broadcasted_iota(jnp.int32, (512, 8), 1)
    oh0 = (lane == a0).astype(jnp.float32)
    oh1 = (lane == a1).astype(jnp.float32)
    oh2 = (lane == a2).astype(jnp.float32)
    g0 = _exact_sel(oh0, be_ref[0, 0], be_ref[1, 0], be_ref[2, 0])
    g1 = _exact_sel(oh1, be_ref[0, 1], be_ref[1, 1], be_ref[2, 1])
    g2 = _exact_sel(oh2, be_ref[0, 2], be_ref[1, 2], be_ref[2, 2])
    o_ref[...] = g0 + g1 + g2


def _t3_build(bond_emb3):
    return pl.pallas_call(
        _t3_body,
        out_shape=jax.ShapeDtypeStruct((512, EMB), jnp.float32),
    )(bond_emb3)


def _c3_body(ea_ref, o_ref):
    o_ref[...] = ea_ref[0] * 64 + ea_ref[1] * 8 + ea_ref[2]


def _c3_build(edge_attr):
    eb = E // 128
    ea3 = edge_attr.T.reshape(3, eb, 128)
    out = pl.pallas_call(
        _c3_body,
        out_shape=jax.ShapeDtypeStruct((eb, 128), jnp.int32),
    )(ea3)
    return out.reshape(E)


def _pre_body(nper, n, with_sg, with_vt, hg_ref, *refs):
    if with_sg:
        sg_ref = refs[0]
        refs = refs[1:]
    vn_ref = refs[0]
    hin_ref = refs[1]
    vt_ref = refs[2] if with_vt else None
    i = pl.program_id(0)
    x = hg_ref[...]
    if with_sg:
        x = x * sg_ref[...]
    rows = i * 512 + jax.lax.broadcasted_iota(jnp.int32, (512, 1), 0)
    vnsel = jnp.zeros((512, EMB), jnp.float32)
    masks = []
    for g in range(B):
        m = (rows >= g * nper) & (rows < (g + 1) * nper)
        masks.append(m)
        vnsel = vnsel + jnp.where(m, vn_ref[g:g + 1, :], 0.0)
    h_in = x + vnsel
    hin_ref[...] = h_in
    if with_vt:
        @pl.when(i == 0)
        def _():
            vt_ref[...] = jnp.zeros((B, EMB), jnp.float32)
        for g in range(B):
            s = jnp.sum(jnp.where(masks[g], h_in, 0.0), axis=0, keepdims=True)
            vt_ref[g:g + 1, :] = vt_ref[g:g + 1, :] + s


def _pre(hg, sg, vnB, nper, with_vt):
    npr = hg.shape[0]
    with_sg = sg is not None
    ins = [hg] + ([sg] if with_sg else []) + [vnB]
    in_specs = [pl.BlockSpec((512, EMB), lambda i: (i, 0))]
    if with_sg:
        in_specs.append(pl.BlockSpec((512, EMB), lambda i: (i, 0)))
    in_specs.append(pl.BlockSpec((B, EMB), lambda i: (0, 0)))
    out_shapes = [jax.ShapeDtypeStruct((npr, EMB), jnp.float32)]
    out_specs = [pl.BlockSpec((512, EMB), lambda i: (i, 0))]
    if with_vt:
        out_shapes.append(jax.ShapeDtypeStruct((B, EMB), jnp.float32))
        out_specs.append(pl.BlockSpec((B, EMB), lambda i: (0, 0)))
    res = pl.pallas_call(
        functools.partial(_pre_body, nper, npr, with_sg, with_vt),
        grid=(npr // 512,),
        in_specs=in_specs,
        out_specs=out_specs,
        out_shape=out_shapes,
    )(*ins)
    return (res[0], res[1]) if with_vt else (res[0], None)


def _mlp_body(with_score, hin_ref, aggp_ref, degp_ref, epsb_ref,
              w1_ref, b1_ref, g1_ref, bb1_ref,
              w2_ref, b2_ref, g2_ref, bb2_ref, sagw_ref,
              h_ref, xw_ref, dinv_ref, u_ref, last_relu):
    agg = aggp_ref[0] + aggp_ref[1]
    z = epsb_ref[...] * hin_ref[...] + agg
    z = _bn(jnp.dot(z, w1_ref[...], preferred_element_type=jnp.float32)
            + b1_ref[...], g1_ref[...], bb1_ref[...])
    z = jnp.maximum(z, 0.0)
    h = jnp.dot(z, w2_ref[...], preferred_element_type=jnp.float32) + b2_ref[...]
    h = _bn(h, g2_ref[...], bb2_ref[...])
    if last_relu:
        h = jnp.maximum(h, 0.0)
    h_ref[...] = h
    if with_score:
        xwf = jnp.dot(h, sagw_ref[...], preferred_element_type=jnp.float32)
        xwc = xwf[:, 0:1]
        xw_ref[...] = jnp.broadcast_to(xwc, (512, 16))
        deg = degp_ref[0] + degp_ref[1] + 1.0
        dinv = jnp.exp(-0.5 * jnp.log(deg))
        dinv_ref[...] = dinv
        u_ref[...] = dinv * xwc
    else:
        z16 = jnp.zeros((512, 16), jnp.float32)
        xw_ref[...] = z16
        dinv_ref[...] = z16
        u_ref[...] = z16


def _mlp(hin, aggp, degp, epsb, w1, b1, g1, bb1, w2, b2, g2, bb2, sagwp,
         with_score, last_relu):
    npr = hin.shape[0]
    in_specs = [
        pl.BlockSpec((512, EMB), lambda i: (i, 0)),
        pl.BlockSpec((2, 512, EMB), lambda i: (0, i, 0)),
        pl.BlockSpec((2, 512, 16), lambda i: (0, i, 0)),
        pl.BlockSpec((1, EMB), lambda i: (0, 0)),
        pl.BlockSpec((EMB, EMB), lambda i: (0, 0)),
        pl.BlockSpec((1, EMB), lambda i: (0, 0)),
        pl.BlockSpec((1, EMB), lambda i: (0, 0)),
        pl.BlockSpec((1, EMB), lambda i: (0, 0)),
        pl.BlockSpec((EMB, EMB), lambda i: (0, 0)),
        pl.BlockSpec((1, EMB), lambda i: (0, 0)),
        pl.BlockSpec((1, EMB), lambda i: (0, 0)),
        pl.BlockSpec((1, EMB), lambda i: (0, 0)),
        pl.BlockSpec((EMB, EMB), lambda i: (0, 0)),
    ]
    out_shapes = [jax.ShapeDtypeStruct((npr, EMB), jnp.float32),
                  jax.ShapeDtypeStruct((npr, 16), jnp.float32),
                  jax.ShapeDtypeStruct((npr, 16), jnp.float32),
                  jax.ShapeDtypeStruct((npr, 16), jnp.float32)]
    out_specs = [pl.BlockSpec((512, EMB), lambda i: (i, 0)),
                 pl.BlockSpec((512, 16), lambda i: (i, 0)),
                 pl.BlockSpec((512, 16), lambda i: (i, 0)),
                 pl.BlockSpec((512, 16), lambda i: (i, 0))]

    def body(hin_ref, aggp_ref, degp_ref, epsb_ref, w1_ref, b1_ref, g1_ref,
             bb1_ref, w2_ref, b2_ref, g2_ref, bb2_ref, sagw_ref,
             h_ref, xw_ref, dinv_ref, u_ref):
        _mlp_body(with_score, hin_ref, aggp_ref, degp_ref, epsb_ref,
                  w1_ref, b1_ref, g1_ref, bb1_ref, w2_ref, b2_ref, g2_ref,
                  bb2_ref, sagw_ref, h_ref, xw_ref, dinv_ref, u_ref,
                  last_relu)

    return pl.pallas_call(
        body,
        grid=(npr // 512,),
        in_specs=in_specs,
        out_specs=out_specs,
        out_shape=out_shapes,
    )(hin, aggp, degp, epsb, w1, b1, g1, bb1, w2, b2, g2, bb2, sagwp)


def _vn_body(vts_ref, vnp_ref, w1_ref, b1_ref, g1_ref, bb1_ref,
             w2_ref, b2_ref, g2_ref, bb2_ref, o_ref):
    vt = vts_ref[...] + vnp_ref[...]
    t = _bn(jnp.dot(vt, w1_ref[...], preferred_element_type=jnp.float32)
            + b1_ref[...], g1_ref[...], bb1_ref[...])
    t = jnp.maximum(t, 0.0)
    t = _bn(jnp.dot(t, w2_ref[...], preferred_element_type=jnp.float32)
            + b2_ref[...], g2_ref[...], bb2_ref[...])
    o_ref[...] = jnp.maximum(t, 0.0)


def _vn_mlp(vtsum, vn_prev, w1, b1, g1, bb1, w2, b2, g2, bb2):
    return pl.pallas_call(
        _vn_body,
        out_shape=jax.ShapeDtypeStruct((B, EMB), jnp.float32),
    )(vtsum, vn_prev, w1.reshape(EMB, EMB), b1.reshape(1, EMB),
      g1.reshape(1, EMB), bb1.reshape(1, EMB), w2.reshape(EMB, EMB),
      b2.reshape(1, EMB), g2.reshape(1, EMB), bb2.reshape(1, EMB))


def _scorefin_body(scp_ref, dinv_ref, xw_ref, sagb_ref, sc_ref, sco_ref):
    sca = scp_ref[0] + scp_ref[1]
    dinv = dinv_ref[...]
    sc = dinv * sca + (dinv * dinv) * xw_ref[...] + sagb_ref[...]
    sc_ref[...] = sc
    sco_ref[...] = jnp.tanh(sc)


def _scorefin(scp, dinv16, xw16, sagb):
    npr = dinv16.shape[0]
    return pl.pallas_call(
        _scorefin_body,
        grid=(npr // 512,),
        in_specs=[pl.BlockSpec((2, 512, 16), lambda i: (0, i, 0)),
                  pl.BlockSpec((512, 16), lambda i: (i, 0)),
                  pl.BlockSpec((512, 16), lambda i: (i, 0)),
                  pl.BlockSpec((1, 16), lambda i: (0, 0))],
        out_specs=[pl.BlockSpec((512, 16), lambda i: (i, 0)),
                   pl.BlockSpec((512, 16), lambda i: (i, 0))],
        out_shape=[jax.ShapeDtypeStruct((npr, 16), jnp.float32),
                   jax.ShapeDtypeStruct((npr, 16), jnp.float32)],
    )(scp, dinv16, xw16, sagb)


def _rank_body(npad, k, sbg3_ref, st3_ref, o_ref):
    g = pl.program_id(0)
    ib = pl.program_id(1)
    icc = st3_ref[...]
    ohg = (jax.lax.broadcasted_iota(jnp.int32, (8, 8), 1) == g)
    for q in range(4):
        ic = jnp.sum(jnp.where(ohg, icc[q], 0.0), axis=1, keepdims=True)
        iidx = (ib * 32 + q * 8
                + jax.lax.broadcasted_iota(jnp.int32, (8, 1), 0))
        cnt = jnp.zeros((8, 128), jnp.int32)
        for jr in range(npad // 128):
            sl = sbg3_ref[:, jr, :]
            jidx = jr * 128 + jax.lax.broadcasted_iota(jnp.int32, (1, 128), 1)
            gt = sl > ic
            tie = (sl == ic) & (jidx < iidx)
            cnt = cnt + (gt | tie).astype(jnp.int32)
        rank = jnp.sum(cnt, axis=1, keepdims=True)
        inv = jnp.where(rank < k, g * k + rank, -1)
        o_ref[0, 0, q] = jnp.broadcast_to(inv, (8, 8))


def _rank(s_bg, npad, nper, k):
    sbg3 = s_bg.reshape(B, npad // 128, 128)
    st3 = s_bg.T.reshape(npad // 8, 8, B)
    out4 = pl.pallas_call(
        functools.partial(_rank_body, npad, k),
        grid=(B, npad // 32),
        in_specs=[pl.BlockSpec((1, npad // 128, 128), lambda g, ib: (g, 0, 0)),
                  pl.BlockSpec((4, 8, 8), lambda g, ib: (ib, 0, 0))],
        out_specs=pl.BlockSpec((1, 1, 4, 8, 8), lambda g, ib: (g, ib, 0, 0, 0)),
        out_shape=jax.ShapeDtypeStruct((B, npad // 32, 4, 8, 8), jnp.int32),
    )(sbg3, st3)
    inv_bg = out4[:, :, :, :, 0].reshape(B, npad)[:, :nper]
    return inv_bg.reshape(B * nper)


def kernel(x, edge_index, edge_attr, batch, atom_emb, bond_emb, vn0,
           gin_W1, gin_b1, gin_bn1_g, gin_bn1_b, gin_W2, gin_b2, gin_eps,
           bn_g, bn_b, vn_W1, vn_b1, vn_bn1_g, vn_bn1_b, vn_W2, vn_b2,
           vn_bn2_g, vn_bn2_b, sag_W, sag_b):
    NP0 = NP_L[0]
    xp = jnp.pad(x.astype(jnp.int32), ((0, NP0 - N0), (0, 16 - 9)))
    h0p = _atom_encode(xp, _split3(atom_emb))
    T3 = _t3_build(_split3(bond_emb))
    c3 = _c3_build(edge_attr.astype(jnp.int32))
    sagwp = jnp.pad(sag_W, ((0, 0), (0, EMB - 1)))
    sagb16 = jnp.broadcast_to(sag_b.reshape(1, 1), (1, 16))

    vn = jnp.broadcast_to(vn0, (B, EMB))
    src = edge_index[0].astype(jnp.int32)
    dst = edge_index[1].astype(jnp.int32)

    hg = h0p
    sgb = None
    h_list = []
    b_list = [jnp.repeat(jnp.arange(B, dtype=jnp.int32), NPER0)]
    h_last = None

    for layer in range(NUM_LAYERS):
        N = LAYER_N[layer]
        NPc = NP_L[layer]
        nper = LAYER_NPER[layer]
        with_score = layer < NUM_LAYERS - 1

        h_in, vtsum = _pre(hg, sgb, vn, nper, with_vt=with_score)
        h_list.append(h_in[:N])

        hin_n = h_in[:N]
        eattr = T3[c3]
        msg = jnp.maximum(hin_n[src] + eattr, 0.0)
        agg = jax.ops.segment_sum(msg, dst, num_segments=NPc)
        aggp = jnp.stack([agg, jnp.zeros_like(agg)])
        if with_score:
            degc = jax.ops.segment_sum(jnp.ones((E,), jnp.float32), dst,
                                       num_segments=NPc)
            degp = jnp.zeros((2, NPc, 16), jnp.float32
                             ).at[0, :, :].set(degc[:, None])
        else:
            degp = jnp.zeros((2, NPc, 16), jnp.float32)

        epsb = jnp.broadcast_to((1.0 + gin_eps[layer]).reshape(1, 1),
                                (1, EMB))
        h, xw16, dinv16, u16 = _mlp(
            h_in, aggp, degp, epsb,
            gin_W1[layer], gin_b1[layer].reshape(1, EMB),
            gin_bn1_g[layer].reshape(1, EMB), gin_bn1_b[layer].reshape(1, EMB),
            gin_W2[layer], gin_b2[layer].reshape(1, EMB),
            bn_g[layer].reshape(1, EMB), bn_b[layer].reshape(1, EMB),
            sagwp, with_score=with_score, last_relu=with_score)

        if not with_score:
            h_last = h[:N]
            b_list.append(b_list[-1])
            break

        vn = _vn_mlp(vtsum, vn,
                     vn_W1[layer], vn_b1[layer], vn_bn1_g[layer],
                     vn_bn1_b[layer], vn_W2[layer], vn_b2[layer],
                     vn_bn2_g[layer], vn_bn2_b[layer])

        u = u16[:, 0]
        valid = dst < N
        contrib = jnp.where(valid, u[jnp.minimum(src, N - 1)], 0.0)
        scagg = jax.ops.segment_sum(contrib, jnp.minimum(dst, NPc - 1),
                                    num_segments=NPc)
        scp = jnp.zeros((2, NPc, 16), jnp.float32).at[0, :, 0].set(scagg)

        sc16, score16 = _scorefin(scp, dinv16, xw16, sagb16)
        score16 = jnp.tanh(sc16)

        k = LAYER_K[layer]
        Nnew = B * k
        NPnew = NP_L[layer + 1]
        npad = 128 * ((nper + 127) // 128)
        s_bg = jnp.full((B, npad), NEG_BIG, jnp.float32
                        ).at[:, :nper].set(score16[:N, 0].reshape(B, nper))
        inv_n = _rank(s_bg, npad, nper, k)
        inv = jnp.pad(inv_n, (0, NPc - N), constant_values=-1)

        perm_t = jnp.where(inv_n >= 0, inv_n, Nnew)
        perm = jnp.zeros((Nnew + 1,), jnp.int32).at[perm_t].set(
            jnp.arange(N, dtype=jnp.int32))[:Nnew]
        permp = jnp.pad(perm, (0, NPnew - Nnew))
        hg = h[permp]
        score_n = score16[:N, 0]
        sgb = jnp.broadcast_to(score_n[permp][:, None], (NPnew, EMB))
        vr = inv[src]
        vc = inv[dst]
        src = jnp.where(vr >= 0, vr, 0)
        dst = jnp.where((vr >= 0) & (vc >= 0), vc, Nnew)
        b_list.append(jnp.repeat(jnp.arange(B, dtype=jnp.int32), k))

    return tuple(h_list) + (h_last,) + tuple(b_list)
